# baseline (speedup 1.0000x reference)
import jax
import jax.numpy as jnp
from jax.experimental import pallas as pl
from jax.experimental.pallas import tpu as pltpu

TL = 1024
NBUF = 4
PREF = 3
NST = 2


def _bag_kernel(total_ref, bag_ref, k_ref, last_ref, n_ref,
                x_ref, w1_ref, b1_ref, w2_ref, b2_ref, out_ref,
                buf_ref, acc_ref, pool_ref, sems):
    total = total_ref[0]
    d = x_ref.shape[2]
    cw = d // NST

    def issue(t):
        @pl.when(t < total)
        def _():
            b = bag_ref[t]
            k = k_ref[t]
            slot = jax.lax.rem(t, NBUF)
            pltpu.make_async_copy(
                x_ref.at[b, pl.ds(k * TL, TL), :],
                buf_ref.at[slot],
                sems.at[slot],
            ).start()

    for i in range(PREF):
        issue(i)

    def body(t, _):
        b = bag_ref[t]
        k = k_ref[t]
        n = n_ref[b]
        slot = jax.lax.rem(t, NBUF)
        pltpu.make_async_copy(
            x_ref.at[b, pl.ds(k * TL, TL), :],
            buf_ref.at[slot],
            sems.at[slot],
        ).wait()
        issue(t + PREF)

        xb = buf_ref[slot].astype(jnp.bfloat16)
        row = k * TL + jax.lax.broadcasted_iota(jnp.int32, (TL, 1), 0)
        ok = row < n
        parts = []
        for h in range(NST):
            yh = jnp.dot(xb, w1_ref[:, h * cw:(h + 1) * cw],
                         preferred_element_type=jnp.float32)
            yh = jnp.maximum(yh + b1_ref[:, h * cw:(h + 1) * cw], 0.0)
            yh = jnp.where(ok, yh, 0.0)
            parts.append(jnp.sum(yh, axis=0, keepdims=True))
        s = jnp.concatenate(parts, axis=1)

        @pl.when(k == 0)
        def init():
            acc_ref[0:1, :] = s

        @pl.when(k != 0)
        def add():
            acc_ref[0:1, :] = acc_ref[0:1, :] + s

        @pl.when(last_ref[t] == 1)
        def finalize():
            pool_ref[pl.ds(b, 1), :] = acc_ref[0:1, :] / n.astype(jnp.float32)

        return ()

    jax.lax.fori_loop(0, total, body, (), unroll=False)

    out = jnp.dot(pool_ref[...], w2_ref[...],
                  preferred_element_type=jnp.float32)
    out_ref[...] = out + b2_ref[...]


def kernel(x, n_instances, W1, b1, W2, b2):
    B, L, D = x.shape
    DO = W2.shape[1]
    max_tiles = B * (L // TL)

    n = n_instances.astype(jnp.int32)
    tiles = (n + TL - 1) // TL
    cum = jnp.cumsum(tiles)
    total = cum[-1:].astype(jnp.int32)
    t_idx = jnp.arange(max_tiles, dtype=jnp.int32)
    bag = jnp.searchsorted(cum, t_idx, side="right").astype(jnp.int32)
    bag = jnp.minimum(bag, B - 1)
    k = t_idx - (cum - tiles)[bag]
    is_last = (k == tiles[bag] - 1).astype(jnp.int32)

    grid_spec = pltpu.PrefetchScalarGridSpec(
        num_scalar_prefetch=5,
        grid=(1,),
        in_specs=[
            pl.BlockSpec(memory_space=pltpu.MemorySpace.HBM),
            pl.BlockSpec((D, D), lambda i, *refs: (0, 0)),
            pl.BlockSpec((1, D), lambda i, *refs: (0, 0)),
            pl.BlockSpec((D, DO), lambda i, *refs: (0, 0)),
            pl.BlockSpec((1, DO), lambda i, *refs: (0, 0)),
        ],
        out_specs=pl.BlockSpec((B, DO), lambda i, *refs: (0, 0)),
        scratch_shapes=[
            pltpu.VMEM((NBUF, TL, D), jnp.float32),
            pltpu.VMEM((8, D), jnp.float32),
            pltpu.VMEM((B, D), jnp.float32),
            pltpu.SemaphoreType.DMA((NBUF,)),
        ],
    )

    return pl.pallas_call(
        _bag_kernel,
        grid_spec=grid_spec,
        out_shape=jax.ShapeDtypeStruct((B, DO), jnp.float32),
        compiler_params=pltpu.CompilerParams(
            dimension_semantics=("arbitrary",),
        ),
    )(total, bag, k, is_last, n,
      x, W1.astype(jnp.bfloat16), b1.reshape(1, D), W2, b2.reshape(1, DO))


# speedup vs baseline: 1.0681x; 1.0681x over previous
import jax
import jax.numpy as jnp
from jax.experimental import pallas as pl
from jax.experimental.pallas import tpu as pltpu

TL = 512
NBUF = 4
PREF = 3
NST = 4


def _bag_kernel(total_ref, bag_ref, k_ref, last_ref, n_ref,
                x_ref, w1_ref, b1_ref, w2_ref, b2_ref, out_ref,
                buf_ref, acc_ref, pool_ref, sems):
    total = total_ref[0]
    d = x_ref.shape[2]
    cw = d // NST

    def issue(t):
        @pl.when(t < total)
        def _():
            b = bag_ref[t]
            k = k_ref[t]
            slot = jax.lax.rem(t, NBUF)
            pltpu.make_async_copy(
                x_ref.at[b, pl.ds(k * TL, TL), :],
                buf_ref.at[slot],
                sems.at[slot],
            ).start()

    for i in range(PREF):
        issue(i)

    def body(t, _):
        b = bag_ref[t]
        k = k_ref[t]
        n = n_ref[b]
        slot = jax.lax.rem(t, NBUF)
        pltpu.make_async_copy(
            x_ref.at[b, pl.ds(k * TL, TL), :],
            buf_ref.at[slot],
            sems.at[slot],
        ).wait()
        issue(t + PREF)

        xb = buf_ref[slot].astype(jnp.bfloat16)
        row = k * TL + jax.lax.broadcasted_iota(jnp.int32, (TL, 1), 0)
        ok = row < n
        parts = []
        for h in range(NST):
            yh = jnp.dot(xb, w1_ref[:, h * cw:(h + 1) * cw],
                         preferred_element_type=jnp.float32)
            yh = jnp.maximum(yh + b1_ref[:, h * cw:(h + 1) * cw], 0.0)
            yh = jnp.where(ok, yh, 0.0)
            parts.append(jnp.sum(yh, axis=0, keepdims=True))
        s = jnp.concatenate(parts, axis=1)

        @pl.when(k == 0)
        def init():
            acc_ref[0:1, :] = s

        @pl.when(k != 0)
        def add():
            acc_ref[0:1, :] = acc_ref[0:1, :] + s

        @pl.when(last_ref[t] == 1)
        def finalize():
            pool_ref[pl.ds(b, 1), :] = acc_ref[0:1, :] / n.astype(jnp.float32)

        return ()

    jax.lax.fori_loop(0, total, body, (), unroll=False)

    out = jnp.dot(pool_ref[...], w2_ref[...],
                  preferred_element_type=jnp.float32)
    out_ref[...] = out + b2_ref[...]


def kernel(x, n_instances, W1, b1, W2, b2):
    B, L, D = x.shape
    DO = W2.shape[1]
    max_tiles = B * (L // TL)

    n = n_instances.astype(jnp.int32)
    tiles = (n + TL - 1) // TL
    cum = jnp.cumsum(tiles)
    total = cum[-1:].astype(jnp.int32)
    t_idx = jnp.arange(max_tiles, dtype=jnp.int32)
    bag = jnp.searchsorted(cum, t_idx, side="right").astype(jnp.int32)
    bag = jnp.minimum(bag, B - 1)
    k = t_idx - (cum - tiles)[bag]
    is_last = (k == tiles[bag] - 1).astype(jnp.int32)

    grid_spec = pltpu.PrefetchScalarGridSpec(
        num_scalar_prefetch=5,
        grid=(1,),
        in_specs=[
            pl.BlockSpec(memory_space=pltpu.MemorySpace.HBM),
            pl.BlockSpec((D, D), lambda i, *refs: (0, 0)),
            pl.BlockSpec((1, D), lambda i, *refs: (0, 0)),
            pl.BlockSpec((D, DO), lambda i, *refs: (0, 0)),
            pl.BlockSpec((1, DO), lambda i, *refs: (0, 0)),
        ],
        out_specs=pl.BlockSpec((B, DO), lambda i, *refs: (0, 0)),
        scratch_shapes=[
            pltpu.VMEM((NBUF, TL, D), jnp.float32),
            pltpu.VMEM((8, D), jnp.float32),
            pltpu.VMEM((B, D), jnp.float32),
            pltpu.SemaphoreType.DMA((NBUF,)),
        ],
    )

    return pl.pallas_call(
        _bag_kernel,
        grid_spec=grid_spec,
        out_shape=jax.ShapeDtypeStruct((B, DO), jnp.float32),
        compiler_params=pltpu.CompilerParams(
            dimension_semantics=("arbitrary",),
        ),
    )(total, bag, k, is_last, n,
      x, W1.astype(jnp.bfloat16), b1.reshape(1, D), W2, b2.reshape(1, DO))
